# baseline (speedup 1.0000x reference)
import functools

import jax
import jax.numpy as jnp
from jax import lax
from jax.experimental import pallas as pl
from jax.experimental.pallas import tpu as pltpu
from jax.experimental.pallas import tpu_sc as plsc

NC = 2
NS = 16
NW = NC * NS
L = 16


_SC_PARAMS = pltpu.CompilerParams(needs_layout_passes=False)


def _vmesh():
    return plsc.VectorSubcoreMesh(core_axis_name="c", subcore_axis_name="s",
                                  num_cores=NC, num_subcores=NS)


def _wid():
    return lax.axis_index("s") * NC + lax.axis_index("c")


def _deg_kernel(E, NP):
    EW = E // NW

    @functools.partial(
        pl.kernel,
        out_type=jax.ShapeDtypeStruct((NW, NP), jnp.float32),
        mesh=_vmesh(),
        compiler_params=_SC_PARAMS,
        scratch_types=[
            pltpu.VMEM((EW,), jnp.int32),
            pltpu.VMEM((EW,), jnp.float32),
            pltpu.VMEM((NP,), jnp.float32),
        ],
    )
    def k(col_hbm, w_hbm, degp_hbm, col_v, w_v, deg_v):
        wid = _wid()
        base = wid * EW
        pltpu.sync_copy(col_hbm.at[pl.ds(base, EW)], col_v)
        pltpu.sync_copy(w_hbm.at[pl.ds(base, EW)], w_v)
        z = jnp.zeros((L,), jnp.float32)

        def zb(i, c):
            deg_v[pl.ds(i * L, L)] = z
            return c
        lax.fori_loop(0, NP // L, zb, 0, unroll=4)

        def eb(i, c):
            s = pl.ds(i * L, L)
            plsc.addupdate_scatter(deg_v, [col_v[s]], w_v[s])
            return c
        lax.fori_loop(0, EW // L, eb, 0, unroll=4)
        pltpu.sync_copy(deg_v, degp_hbm.at[wid])

    return k


def _dis_kernel(NP):
    SL = NP // NW

    @functools.partial(
        pl.kernel,
        out_type=(jax.ShapeDtypeStruct((NP,), jnp.float32),
                  jax.ShapeDtypeStruct((NP,), jnp.float32)),
        mesh=_vmesh(),
        compiler_params=_SC_PARAMS,
        scratch_types=[
            pltpu.VMEM((NW, SL), jnp.float32),
            pltpu.VMEM((SL,), jnp.float32),
            pltpu.VMEM((SL,), jnp.float32),
        ],
    )
    def k(degp_hbm, dis_hbm, dis2_hbm, pb_v, dis_v, dis2_v):
        wid = _wid()
        base = wid * SL
        pltpu.sync_copy(degp_hbm.at[:, pl.ds(base, SL)], pb_v)

        def rs(i, c):
            s = pl.ds(i * L, L)
            a = pb_v[0, s]
            for p in range(1, NW):
                a = a + pb_v[p, s]
            d = a + 1.0
            y = plsc.bitcast(jnp.int32(0x5F3759DF)
                             - (plsc.bitcast(d, jnp.int32) >> 1), jnp.float32)
            hf = 0.5 * d
            y = y * (1.5 - hf * y * y)
            y = y * (1.5 - hf * y * y)
            y = y * (1.5 - hf * y * y)
            dis_v[s] = y
            dis2_v[s] = y * y
            return c
        lax.fori_loop(0, SL // L, rs, 0, unroll=2)
        pltpu.sync_copy(dis_v, dis_hbm.at[pl.ds(base, SL)])
        pltpu.sync_copy(dis2_v, dis2_hbm.at[pl.ds(base, SL)])

    return k


def _norm_kernel(E, NP):
    EW = E // NW

    @functools.partial(
        pl.kernel,
        out_type=(jax.ShapeDtypeStruct((E,), jnp.float32),
                  jax.ShapeDtypeStruct((E,), jnp.int32)),
        mesh=_vmesh(),
        compiler_params=_SC_PARAMS,
        scratch_types=[
            pltpu.VMEM((NP,), jnp.float32),
            pltpu.VMEM((EW,), jnp.int32),
            pltpu.VMEM((EW,), jnp.int32),
            pltpu.VMEM((EW,), jnp.float32),
            pltpu.VMEM((EW,), jnp.float32),
            pltpu.VMEM((EW,), jnp.int32),
        ],
    )
    def k(dis_hbm, row_hbm, col_hbm, w_hbm, norm_hbm, rc_hbm,
          dis_v, row_v, col_v, w_v, nrm_v, rc_v):
        wid = _wid()
        pltpu.sync_copy(dis_hbm, dis_v)
        base = wid * EW
        pltpu.sync_copy(row_hbm.at[pl.ds(base, EW)], row_v)
        pltpu.sync_copy(col_hbm.at[pl.ds(base, EW)], col_v)
        pltpu.sync_copy(w_hbm.at[pl.ds(base, EW)], w_v)

        def nb(i, c):
            s = pl.ds(i * L, L)
            r = row_v[s]
            cc = col_v[s]
            a = plsc.load_gather(dis_v, [r])
            b = plsc.load_gather(dis_v, [cc])
            nrm_v[s] = a * w_v[s] * b
            rc_v[s] = r | (cc << 16)
            return c
        lax.fori_loop(0, EW // L, nb, 0, unroll=4)
        pltpu.sync_copy(nrm_v, norm_hbm.at[pl.ds(base, EW)])
        pltpu.sync_copy(rc_v, rc_hbm.at[pl.ds(base, EW)])

    return k


def _pack_pairs(he, ho):
    ue = lax.bitcast_convert_type(he, jnp.uint32)
    uo = lax.bitcast_convert_type(ho, jnp.uint32)
    hi = (ue + jnp.uint32(0x8000)) & jnp.uint32(0xFFFF0000)
    lo = (uo + jnp.uint32(0x8000)) >> 16
    return lax.bitcast_convert_type(hi | lo, jnp.int32)


def _mm_T_packed(x, We, Wo):
    n, d = x.shape
    hp = We.shape[1]

    def body(x_ref, we_ref, wo_ref, o_ref):
        he = lax.dot_general(we_ref[...], x_ref[...], (((0,), (1,)), ((), ())),
                             preferred_element_type=jnp.float32)
        ho = lax.dot_general(wo_ref[...], x_ref[...], (((0,), (1,)), ((), ())),
                             preferred_element_type=jnp.float32)
        o_ref[...] = _pack_pairs(he, ho)

    return pl.pallas_call(
        body,
        out_shape=jax.ShapeDtypeStruct((hp, n), jnp.int32),
    )(x, We, Wo)


def _mm_TT_packed(xT, We, Wo):
    d, n = xT.shape
    hp = We.shape[1]

    def body(xT_ref, we_ref, wo_ref, o_ref):
        he = lax.dot_general(we_ref[...], xT_ref[...], (((0,), (0,)), ((), ())),
                             preferred_element_type=jnp.float32)
        ho = lax.dot_general(wo_ref[...], xT_ref[...], (((0,), (0,)), ((), ())),
                             preferred_element_type=jnp.float32)
        o_ref[...] = _pack_pairs(he, ho)

    return pl.pallas_call(
        body,
        out_shape=jax.ShapeDtypeStruct((hp, n), jnp.int32),
    )(xT, We, Wo)


def _unpack_hi(p):
    return plsc.bitcast(p & jnp.int32(-65536), jnp.float32)


def _unpack_lo(p):
    return plsc.bitcast(p << 16, jnp.float32)


def _spmm_kernel(E, N, FT, relu):
    CH = 4000
    NCHK = E // CH
    FP = FT // 2

    scratch = ([pltpu.VMEM((N,), jnp.int32) for _ in range(FP)]
               + [pltpu.VMEM((N,), jnp.float32) for _ in range(FT)]
               + [pltpu.VMEM((N,), jnp.float32),
                  pltpu.VMEM((FT * NW,), jnp.float32)]
               + [pltpu.VMEM((CH,), jnp.int32) for _ in range(2)]
               + [pltpu.VMEM((CH,), jnp.float32) for _ in range(2)]
               + [pltpu.SemaphoreType.DMA for _ in range(2)])

    @functools.partial(
        pl.kernel,
        out_type=jax.ShapeDtypeStruct((FT * NW, N), jnp.float32),
        mesh=_vmesh(),
        compiler_params=_SC_PARAMS,
        scratch_types=scratch,
    )
    def k(hTp_hbm, rc_hbm, nrm_hbm, dis2_hbm, b_hbm, outT_hbm, *scr):
        hv = scr[0:FP]
        ov = scr[FP:FP + FT]
        dis2_v, b_v = scr[FP + FT:FP + FT + 2]
        rc_b = scr[FP + FT + 2:FP + FT + 4]
        nrm_b = scr[FP + FT + 4:FP + FT + 6]
        sem = scr[FP + FT + 6:FP + FT + 8]
        wid = _wid()
        for q in range(FP):
            pltpu.sync_copy(hTp_hbm.at[wid * FP + q], hv[q])
        pltpu.sync_copy(dis2_hbm.at[pl.ds(0, N)], dis2_v)
        pltpu.sync_copy(b_hbm, b_v)

        z = jnp.zeros((L,), jnp.float32)

        @plsc.parallel_loop(0, N, step=L, unroll=4)
        def _zb(i):
            s = pl.ds(i, L)
            for f in range(FT):
                ov[f][s] = z

        def start(g, b):
            base = g * CH
            pltpu.async_copy(rc_hbm.at[pl.ds(base, CH)], rc_b[b], sem[b])
            pltpu.async_copy(nrm_hbm.at[pl.ds(base, CH)], nrm_b[b], sem[b])

        def drain(g, b):
            base = g * CH
            pltpu.make_async_copy(rc_hbm.at[pl.ds(base, CH)], rc_b[b], sem[b]).wait()
            pltpu.make_async_copy(nrm_hbm.at[pl.ds(base, CH)], nrm_b[b], sem[b]).wait()

        start(0, 0)
        start(1, 1)

        def pair(g0, c):
            for b in range(2):
                g = g0 * 2 + b
                drain(g, b)

                @plsc.parallel_loop(0, CH, step=L, unroll=8)
                def _eb(i):
                    s = pl.ds(i, L)
                    p = rc_b[b][s]
                    r = p & jnp.int32(0xFFFF)
                    cc = p >> 16
                    nm = nrm_b[b][s]
                    for q in range(FP):
                        g16 = plsc.load_gather(hv[q], [r])
                        plsc.addupdate_scatter(ov[2 * q], [cc], _unpack_hi(g16) * nm)
                        plsc.addupdate_scatter(ov[2 * q + 1], [cc], _unpack_lo(g16) * nm)

                @pl.when(g + 2 < NCHK)
                def _():
                    start(g + 2, b)
            return c
        lax.fori_loop(0, NCHK // 2, pair, 0)

        OFF = FT * NW // 2
        for q in range(FP):
            f0 = wid * FP + q
            f1 = f0 + OFF
            b0 = plsc.load_gather(b_v, [jnp.full((L,), f0, jnp.int32)])
            b1 = plsc.load_gather(b_v, [jnp.full((L,), f1, jnp.int32)])

            @plsc.parallel_loop(0, N, step=L, unroll=4)
            def _ep(i, q=q, b0=b0, b1=b1):
                s = pl.ds(i, L)
                pv = hv[q][s]
                v0 = ov[2 * q][s] + dis2_v[s] * _unpack_hi(pv) + b0
                v1 = ov[2 * q + 1][s] + dis2_v[s] * _unpack_lo(pv) + b1
                if relu:
                    v0 = jnp.maximum(v0, 0.0)
                    v1 = jnp.maximum(v1, 0.0)
                ov[2 * q][s] = v0
                ov[2 * q + 1][s] = v1

            pltpu.sync_copy(ov[2 * q], outT_hbm.at[f0])
            pltpu.sync_copy(ov[2 * q + 1], outT_hbm.at[f1])

    return k


def _spmm_part_kernel(E, N, FT, EG):
    GF = NW // EG
    F = GF * FT
    EE = E // EG
    CH = 4000
    NCHK = EE // CH

    FP = FT // 2
    scratch = ([pltpu.VMEM((N,), jnp.int32) for _ in range(FP)]
               + [pltpu.VMEM((N,), jnp.float32) for _ in range(FT)]
               + [pltpu.VMEM((CH,), jnp.int32) for _ in range(2)]
               + [pltpu.VMEM((CH,), jnp.float32) for _ in range(2)]
               + [pltpu.SemaphoreType.DMA for _ in range(2)])

    @functools.partial(
        pl.kernel,
        out_type=jax.ShapeDtypeStruct((EG, F, N), jnp.float32),
        mesh=_vmesh(),
        compiler_params=_SC_PARAMS,
        scratch_types=scratch,
    )
    def k(hTp_hbm, rc_hbm, nrm_hbm, out_hbm, *scr):
        hv = scr[0:FP]
        ov = scr[FP:FP + FT]
        rc_b = scr[FP + FT:FP + FT + 2]
        nrm_b = scr[FP + FT + 2:FP + FT + 4]
        sem = scr[FP + FT + 4:FP + FT + 6]
        wid = _wid()
        eg = wid // GF
        fs = wid % GF
        ebase = eg * EE
        for q in range(FP):
            pltpu.sync_copy(hTp_hbm.at[fs * FP + q], hv[q])

        z = jnp.zeros((L,), jnp.float32)

        @plsc.parallel_loop(0, N, step=L, unroll=4)
        def _zb(i):
            s = pl.ds(i, L)
            for f in range(FT):
                ov[f][s] = z

        def start(g, b):
            base = ebase + g * CH
            pltpu.async_copy(rc_hbm.at[pl.ds(base, CH)], rc_b[b], sem[b])
            pltpu.async_copy(nrm_hbm.at[pl.ds(base, CH)], nrm_b[b], sem[b])

        def drain(g, b):
            base = ebase + g * CH
            pltpu.make_async_copy(rc_hbm.at[pl.ds(base, CH)], rc_b[b], sem[b]).wait()
            pltpu.make_async_copy(nrm_hbm.at[pl.ds(base, CH)], nrm_b[b], sem[b]).wait()

        start(0, 0)
        start(1, 1)

        def pair(g0, c):
            for b in range(2):
                g = g0 * 2 + b
                drain(g, b)

                @plsc.parallel_loop(0, CH, step=L, unroll=8)
                def _eb(i):
                    s = pl.ds(i, L)
                    p = rc_b[b][s]
                    r = p & jnp.int32(0xFFFF)
                    cc = p >> 16
                    nm = nrm_b[b][s]
                    for q in range(FP):
                        g16 = plsc.load_gather(hv[q], [r])
                        plsc.addupdate_scatter(ov[2 * q], [cc], _unpack_hi(g16) * nm)
                        plsc.addupdate_scatter(ov[2 * q + 1], [cc], _unpack_lo(g16) * nm)

                @pl.when(g + 2 < NCHK)
                def _():
                    start(g + 2, b)
            return c
        lax.fori_loop(0, NCHK // 2, pair, 0)

        OFF = F // 2
        for q in range(FP):
            pltpu.sync_copy(ov[2 * q], out_hbm.at[eg, fs * FP + q])
            pltpu.sync_copy(ov[2 * q + 1], out_hbm.at[eg, fs * FP + q + OFF])

    return k


def _combine2(part, hTp, dis2n, bcol):
    eg, f, n = part.shape

    def body(p_ref, h_ref, d_ref, b_ref, o_ref):
        s = p_ref[0]
        for g in range(1, eg):
            s = s + p_ref[g]
        hp = h_ref[...]
        he = lax.bitcast_convert_type(
            hp & jnp.int32(-65536), jnp.float32)
        ho = lax.bitcast_convert_type(hp << 16, jnp.float32)
        hfull = jnp.concatenate([he, ho], axis=0)
        o_ref[...] = s + d_ref[...] * hfull + b_ref[...]

    return pl.pallas_call(
        body,
        out_shape=jax.ShapeDtypeStruct((f, n), jnp.float32),
    )(part, hTp, dis2n, bcol)


def kernel(x, edge_index, edge_weight, W1, b1, W2, b2):
    n, d = x.shape
    e = edge_weight.shape[0]
    h = W1.shape[1]
    o = W2.shape[1]
    op = NW
    row = edge_index[0]
    col = edge_index[1]

    sl = ((n + NW - 1) // NW + 127) // 128 * 128
    np_ = NW * sl
    degp = _deg_kernel(e, np_)(col, edge_weight)
    dis, dis2 = _dis_kernel(np_)(degp)
    norm, rcp = _norm_kernel(e, np_)(dis, row, col, edge_weight)
    h1Tp = _mm_T_packed(x, W1[:, :h // 2], W1[:, h // 2:])
    a1T = _spmm_kernel(e, n, h // NW, True)(h1Tp, rcp, norm, dis2, b1)
    W2p = jnp.zeros((h, op), W2.dtype).at[:, :o].set(W2)
    b2p = jnp.zeros((op,), b2.dtype).at[:o].set(b2)
    h2Tp = _mm_TT_packed(a1T, W2p[:, :op // 2], W2p[:, op // 2:])
    part2 = _spmm_part_kernel(e, n, 4, 4)(h2Tp, rcp, norm)
    outT = _combine2(part2, h2Tp, dis2[:n].reshape(1, n), b2p.reshape(op, 1))
    return outT[:o].T


# speedup vs baseline: 1.0498x; 1.0498x over previous
import functools

import jax
import jax.numpy as jnp
from jax import lax
from jax.experimental import pallas as pl
from jax.experimental.pallas import tpu as pltpu
from jax.experimental.pallas import tpu_sc as plsc

NC = 2
NS = 16
NW = NC * NS
L = 16


_SC_PARAMS = pltpu.CompilerParams(needs_layout_passes=False)


def _vmesh():
    return plsc.VectorSubcoreMesh(core_axis_name="c", subcore_axis_name="s",
                                  num_cores=NC, num_subcores=NS)


def _wid():
    return lax.axis_index("s") * NC + lax.axis_index("c")


def _deg_kernel(E, NP):
    EW = E // NW

    @functools.partial(
        pl.kernel,
        out_type=jax.ShapeDtypeStruct((NW, NP), jnp.float32),
        mesh=_vmesh(),
        compiler_params=_SC_PARAMS,
        scratch_types=[
            pltpu.VMEM((EW,), jnp.int32),
            pltpu.VMEM((EW,), jnp.float32),
            pltpu.VMEM((NP,), jnp.float32),
        ],
    )
    def k(col_hbm, w_hbm, degp_hbm, col_v, w_v, deg_v):
        wid = _wid()
        base = wid * EW
        pltpu.sync_copy(col_hbm.at[pl.ds(base, EW)], col_v)
        pltpu.sync_copy(w_hbm.at[pl.ds(base, EW)], w_v)
        z = jnp.zeros((L,), jnp.float32)

        def zb(i, c):
            deg_v[pl.ds(i * L, L)] = z
            return c
        lax.fori_loop(0, NP // L, zb, 0, unroll=4)

        def eb(i, c):
            s = pl.ds(i * L, L)
            plsc.addupdate_scatter(deg_v, [col_v[s]], w_v[s])
            return c
        lax.fori_loop(0, EW // L, eb, 0, unroll=4)
        pltpu.sync_copy(deg_v, degp_hbm.at[wid])

    return k


def _dis_kernel(NP):
    SL = NP // NW

    @functools.partial(
        pl.kernel,
        out_type=(jax.ShapeDtypeStruct((NP,), jnp.float32),
                  jax.ShapeDtypeStruct((NP,), jnp.float32)),
        mesh=_vmesh(),
        compiler_params=_SC_PARAMS,
        scratch_types=[
            pltpu.VMEM((NW, SL), jnp.float32),
            pltpu.VMEM((SL,), jnp.float32),
            pltpu.VMEM((SL,), jnp.float32),
        ],
    )
    def k(degp_hbm, dis_hbm, dis2_hbm, pb_v, dis_v, dis2_v):
        wid = _wid()
        base = wid * SL
        pltpu.sync_copy(degp_hbm.at[:, pl.ds(base, SL)], pb_v)

        def rs(i, c):
            s = pl.ds(i * L, L)
            a = pb_v[0, s]
            for p in range(1, NW):
                a = a + pb_v[p, s]
            d = a + 1.0
            y = plsc.bitcast(jnp.int32(0x5F3759DF)
                             - (plsc.bitcast(d, jnp.int32) >> 1), jnp.float32)
            hf = 0.5 * d
            y = y * (1.5 - hf * y * y)
            y = y * (1.5 - hf * y * y)
            y = y * (1.5 - hf * y * y)
            dis_v[s] = y
            dis2_v[s] = y * y
            return c
        lax.fori_loop(0, SL // L, rs, 0, unroll=2)
        pltpu.sync_copy(dis_v, dis_hbm.at[pl.ds(base, SL)])
        pltpu.sync_copy(dis2_v, dis2_hbm.at[pl.ds(base, SL)])

    return k


def _norm_kernel(E, NP):
    EW = E // NW

    @functools.partial(
        pl.kernel,
        out_type=(jax.ShapeDtypeStruct((E,), jnp.float32),
                  jax.ShapeDtypeStruct((E,), jnp.int32)),
        mesh=_vmesh(),
        compiler_params=_SC_PARAMS,
        scratch_types=[
            pltpu.VMEM((NP,), jnp.float32),
            pltpu.VMEM((EW,), jnp.int32),
            pltpu.VMEM((EW,), jnp.int32),
            pltpu.VMEM((EW,), jnp.float32),
            pltpu.VMEM((EW,), jnp.float32),
            pltpu.VMEM((EW,), jnp.int32),
        ],
    )
    def k(dis_hbm, row_hbm, col_hbm, w_hbm, norm_hbm, rc_hbm,
          dis_v, row_v, col_v, w_v, nrm_v, rc_v):
        wid = _wid()
        pltpu.sync_copy(dis_hbm, dis_v)
        base = wid * EW
        pltpu.sync_copy(row_hbm.at[pl.ds(base, EW)], row_v)
        pltpu.sync_copy(col_hbm.at[pl.ds(base, EW)], col_v)
        pltpu.sync_copy(w_hbm.at[pl.ds(base, EW)], w_v)

        def nb(i, c):
            s = pl.ds(i * L, L)
            r = row_v[s]
            cc = col_v[s]
            a = plsc.load_gather(dis_v, [r])
            b = plsc.load_gather(dis_v, [cc])
            nrm_v[s] = a * w_v[s] * b
            rc_v[s] = r | (cc << 16)
            return c
        lax.fori_loop(0, EW // L, nb, 0, unroll=4)
        pltpu.sync_copy(nrm_v, norm_hbm.at[pl.ds(base, EW)])
        pltpu.sync_copy(rc_v, rc_hbm.at[pl.ds(base, EW)])

    return k


def _pack_pairs(he, ho):
    ue = lax.bitcast_convert_type(he, jnp.uint32)
    uo = lax.bitcast_convert_type(ho, jnp.uint32)
    hi = (ue + jnp.uint32(0x8000)) & jnp.uint32(0xFFFF0000)
    lo = (uo + jnp.uint32(0x8000)) >> 16
    return lax.bitcast_convert_type(hi | lo, jnp.int32)


def _mm_T_packed(x, We, Wo):
    n, d = x.shape
    hp = We.shape[1]

    def body(x_ref, we_ref, wo_ref, o_ref):
        he = lax.dot_general(we_ref[...], x_ref[...], (((0,), (1,)), ((), ())),
                             preferred_element_type=jnp.float32)
        ho = lax.dot_general(wo_ref[...], x_ref[...], (((0,), (1,)), ((), ())),
                             preferred_element_type=jnp.float32)
        o_ref[...] = _pack_pairs(he, ho)

    return pl.pallas_call(
        body,
        out_shape=jax.ShapeDtypeStruct((hp, n), jnp.int32),
    )(x, We, Wo)


def _mm_TT_packed(xT, We, Wo):
    d, n = xT.shape
    hp = We.shape[1]

    def body(xT_ref, we_ref, wo_ref, o_ref):
        he = lax.dot_general(we_ref[...], xT_ref[...], (((0,), (0,)), ((), ())),
                             preferred_element_type=jnp.float32)
        ho = lax.dot_general(wo_ref[...], xT_ref[...], (((0,), (0,)), ((), ())),
                             preferred_element_type=jnp.float32)
        o_ref[...] = _pack_pairs(he, ho)

    return pl.pallas_call(
        body,
        out_shape=jax.ShapeDtypeStruct((hp, n), jnp.int32),
    )(xT, We, Wo)


def _unpack_hi(p):
    return plsc.bitcast(p & jnp.int32(-65536), jnp.float32)


def _unpack_lo(p):
    return plsc.bitcast(p << 16, jnp.float32)


def _spmm_kernel(E, N, FT, relu):
    CH = 4000
    NCHK = E // CH
    FP = FT // 2

    scratch = ([pltpu.VMEM((N,), jnp.int32) for _ in range(FP)]
               + [pltpu.VMEM((N,), jnp.float32) for _ in range(FT)]
               + [pltpu.VMEM((N,), jnp.float32),
                  pltpu.VMEM((FT * NW,), jnp.float32)]
               + [pltpu.VMEM((CH,), jnp.int32) for _ in range(2)]
               + [pltpu.VMEM((CH,), jnp.float32) for _ in range(2)]
               + [pltpu.SemaphoreType.DMA for _ in range(2)])

    @functools.partial(
        pl.kernel,
        out_type=jax.ShapeDtypeStruct((FT * NW, N), jnp.float32),
        mesh=_vmesh(),
        compiler_params=_SC_PARAMS,
        scratch_types=scratch,
    )
    def k(hTp_hbm, rc_hbm, nrm_hbm, dis2_hbm, b_hbm, outT_hbm, *scr):
        hv = scr[0:FP]
        ov = scr[FP:FP + FT]
        dis2_v, b_v = scr[FP + FT:FP + FT + 2]
        rc_b = scr[FP + FT + 2:FP + FT + 4]
        nrm_b = scr[FP + FT + 4:FP + FT + 6]
        sem = scr[FP + FT + 6:FP + FT + 8]
        wid = _wid()
        for q in range(FP):
            pltpu.sync_copy(hTp_hbm.at[wid * FP + q], hv[q])
        pltpu.sync_copy(dis2_hbm.at[pl.ds(0, N)], dis2_v)
        pltpu.sync_copy(b_hbm, b_v)

        z = jnp.zeros((L,), jnp.float32)

        @plsc.parallel_loop(0, N, step=L, unroll=4)
        def _zb(i):
            s = pl.ds(i, L)
            for f in range(FT):
                ov[f][s] = z

        def start(g, b):
            base = g * CH
            pltpu.async_copy(rc_hbm.at[pl.ds(base, CH)], rc_b[b], sem[b])
            pltpu.async_copy(nrm_hbm.at[pl.ds(base, CH)], nrm_b[b], sem[b])

        def drain(g, b):
            base = g * CH
            pltpu.make_async_copy(rc_hbm.at[pl.ds(base, CH)], rc_b[b], sem[b]).wait()
            pltpu.make_async_copy(nrm_hbm.at[pl.ds(base, CH)], nrm_b[b], sem[b]).wait()

        start(0, 0)
        start(1, 1)

        def pair(g0, c):
            for b in range(2):
                g = g0 * 2 + b
                drain(g, b)

                @plsc.parallel_loop(0, CH, step=L, unroll=4)
                def _eb(i):
                    s = pl.ds(i, L)
                    p = rc_b[b][s]
                    r = p & jnp.int32(0xFFFF)
                    cc = p >> 16
                    nm = nrm_b[b][s]
                    for q in range(FP):
                        g16 = plsc.load_gather(hv[q], [r])
                        plsc.addupdate_scatter(ov[2 * q], [cc], _unpack_hi(g16) * nm)
                        plsc.addupdate_scatter(ov[2 * q + 1], [cc], _unpack_lo(g16) * nm)

                @pl.when(g + 2 < NCHK)
                def _():
                    start(g + 2, b)
            return c
        lax.fori_loop(0, NCHK // 2, pair, 0)

        OFF = FT * NW // 2
        for q in range(FP):
            f0 = wid * FP + q
            f1 = f0 + OFF
            b0 = plsc.load_gather(b_v, [jnp.full((L,), f0, jnp.int32)])
            b1 = plsc.load_gather(b_v, [jnp.full((L,), f1, jnp.int32)])

            @plsc.parallel_loop(0, N, step=L, unroll=4)
            def _ep(i, q=q, b0=b0, b1=b1):
                s = pl.ds(i, L)
                pv = hv[q][s]
                v0 = ov[2 * q][s] + dis2_v[s] * _unpack_hi(pv) + b0
                v1 = ov[2 * q + 1][s] + dis2_v[s] * _unpack_lo(pv) + b1
                if relu:
                    v0 = jnp.maximum(v0, 0.0)
                    v1 = jnp.maximum(v1, 0.0)
                ov[2 * q][s] = v0
                ov[2 * q + 1][s] = v1

            pltpu.sync_copy(ov[2 * q], outT_hbm.at[f0])
            pltpu.sync_copy(ov[2 * q + 1], outT_hbm.at[f1])

    return k


def _spmm_part_kernel(E, N, FT, EG, CH=4000):
    GF = NW // EG
    F = GF * FT
    EE = E // EG
    NCHK = EE // CH

    FP = FT // 2
    scratch = ([pltpu.VMEM((N,), jnp.int32) for _ in range(FP)]
               + [pltpu.VMEM((N,), jnp.float32) for _ in range(FT)]
               + [pltpu.VMEM((CH,), jnp.int32) for _ in range(2)]
               + [pltpu.VMEM((CH,), jnp.float32) for _ in range(2)]
               + [pltpu.SemaphoreType.DMA for _ in range(2)])

    @functools.partial(
        pl.kernel,
        out_type=jax.ShapeDtypeStruct((EG, F, N), jnp.float32),
        mesh=_vmesh(),
        compiler_params=_SC_PARAMS,
        scratch_types=scratch,
    )
    def k(hTp_hbm, rc_hbm, nrm_hbm, out_hbm, *scr):
        hv = scr[0:FP]
        ov = scr[FP:FP + FT]
        rc_b = scr[FP + FT:FP + FT + 2]
        nrm_b = scr[FP + FT + 2:FP + FT + 4]
        sem = scr[FP + FT + 4:FP + FT + 6]
        wid = _wid()
        eg = wid // GF
        fs = wid % GF
        ebase = eg * EE
        for q in range(FP):
            pltpu.sync_copy(hTp_hbm.at[fs * FP + q], hv[q])

        z = jnp.zeros((L,), jnp.float32)

        @plsc.parallel_loop(0, N, step=L, unroll=4)
        def _zb(i):
            s = pl.ds(i, L)
            for f in range(FT):
                ov[f][s] = z

        def start(g, b):
            base = ebase + g * CH
            pltpu.async_copy(rc_hbm.at[pl.ds(base, CH)], rc_b[b], sem[b])
            pltpu.async_copy(nrm_hbm.at[pl.ds(base, CH)], nrm_b[b], sem[b])

        def drain(g, b):
            base = ebase + g * CH
            pltpu.make_async_copy(rc_hbm.at[pl.ds(base, CH)], rc_b[b], sem[b]).wait()
            pltpu.make_async_copy(nrm_hbm.at[pl.ds(base, CH)], nrm_b[b], sem[b]).wait()

        start(0, 0)
        start(1, 1)

        def pair(g0, c):
            for b in range(2):
                g = g0 * 2 + b
                drain(g, b)

                @plsc.parallel_loop(0, CH, step=L, unroll=4)
                def _eb(i):
                    s = pl.ds(i, L)
                    p = rc_b[b][s]
                    r = p & jnp.int32(0xFFFF)
                    cc = p >> 16
                    nm = nrm_b[b][s]
                    for q in range(FP):
                        g16 = plsc.load_gather(hv[q], [r])
                        plsc.addupdate_scatter(ov[2 * q], [cc], _unpack_hi(g16) * nm)
                        plsc.addupdate_scatter(ov[2 * q + 1], [cc], _unpack_lo(g16) * nm)

                @pl.when(g + 2 < NCHK)
                def _():
                    start(g + 2, b)
            return c
        lax.fori_loop(0, NCHK // 2, pair, 0)

        OFF = F // 2
        for q in range(FP):
            pltpu.sync_copy(ov[2 * q], out_hbm.at[eg, fs * FP + q])
            pltpu.sync_copy(ov[2 * q + 1], out_hbm.at[eg, fs * FP + q + OFF])

    return k


def _combine1_mm2(part, hTp, dis2n, bcol, We, Wo):
    eg, f, n = part.shape
    hp2 = We.shape[1]

    def body(p_ref, h_ref, d_ref, b_ref, we_ref, wo_ref, o_ref):
        s = p_ref[0]
        for g in range(1, eg):
            s = s + p_ref[g]
        hp = h_ref[...]
        he = lax.bitcast_convert_type(hp & jnp.int32(-65536), jnp.float32)
        ho = lax.bitcast_convert_type(hp << 16, jnp.float32)
        hfull = jnp.concatenate([he, ho], axis=0)
        a1 = jnp.maximum(s + d_ref[...] * hfull + b_ref[...], 0.0)
        h2e = lax.dot_general(we_ref[...], a1, (((0,), (0,)), ((), ())),
                              preferred_element_type=jnp.float32)
        h2o = lax.dot_general(wo_ref[...], a1, (((0,), (0,)), ((), ())),
                              preferred_element_type=jnp.float32)
        o_ref[...] = _pack_pairs(h2e, h2o)

    return pl.pallas_call(
        body,
        out_shape=jax.ShapeDtypeStruct((hp2, n), jnp.int32),
    )(part, hTp, dis2n, bcol, We, Wo)


def _combine2(part, hTp, dis2n, bcol):
    eg, f, n = part.shape

    def body(p_ref, h_ref, d_ref, b_ref, o_ref):
        s = p_ref[0]
        for g in range(1, eg):
            s = s + p_ref[g]
        hp = h_ref[...]
        he = lax.bitcast_convert_type(
            hp & jnp.int32(-65536), jnp.float32)
        ho = lax.bitcast_convert_type(hp << 16, jnp.float32)
        hfull = jnp.concatenate([he, ho], axis=0)
        o_ref[...] = s + d_ref[...] * hfull + b_ref[...]

    return pl.pallas_call(
        body,
        out_shape=jax.ShapeDtypeStruct((f, n), jnp.float32),
    )(part, hTp, dis2n, bcol)


def kernel(x, edge_index, edge_weight, W1, b1, W2, b2):
    n, d = x.shape
    e = edge_weight.shape[0]
    h = W1.shape[1]
    o = W2.shape[1]
    op = NW
    row = edge_index[0]
    col = edge_index[1]

    sl = ((n + NW - 1) // NW + 127) // 128 * 128
    np_ = NW * sl
    degp = _deg_kernel(e, np_)(col, edge_weight)
    dis, dis2 = _dis_kernel(np_)(degp)
    norm, rcp = _norm_kernel(e, np_)(dis, row, col, edge_weight)
    h1Tp = _mm_T_packed(x, W1[:, :h // 2], W1[:, h // 2:])
    part1 = _spmm_part_kernel(e, n, 8, 2, 1600)(h1Tp, rcp, norm)
    W2p = jnp.zeros((h, op), W2.dtype).at[:, :o].set(W2)
    b2p = jnp.zeros((op,), b2.dtype).at[:o].set(b2)
    h2Tp = _combine1_mm2(part1, h1Tp, dis2[:n].reshape(1, n),
                         b1.reshape(h, 1),
                         W2p[:, :op // 2], W2p[:, op // 2:])
    part2 = _spmm_part_kernel(e, n, 4, 4)(h2Tp, rcp, norm)
    outT = _combine2(part2, h2Tp, dis2[:n].reshape(1, n), b2p.reshape(op, 1))
    return outT[:o].T
